# baseline (speedup 1.0000x reference)
import functools

import jax
import jax.numpy as jnp
from jax import lax
from jax.experimental import pallas as pl
from jax.experimental.pallas import tpu as pltpu, tpu_sc as plsc

L = 1025
S = 32
NB = 32
D = 64
NLANE = 16
NV = D // NLANE
CLIP = 14
WROWS = 1016
HALF = 512
CROWS = 24
NHEAD = 15
MAXQ = 14


def _sl(k):
    return pl.ds(k * NLANE, NLANE)


def _emit_plan(bi):
    plan = []
    lext = max(0, 32 * (bi - 15))
    j = 8
    while lext > 0:
        plan.append((j, 32, 'L', 0))
        j += 32
        lext -= 32
    j_end = min(1024, 479 + 32 * bi)
    q0 = j + 472 - 32 * bi
    q1 = j_end + 472 - 32 * bi
    if q0 <= min(q1, HALF - 1):
        n = min(q1, HALF - 1) - q0 + 1
        plan.append((j, n, 'A', q0))
        j += n
    if max(q0, HALF) <= q1:
        n = q1 - max(q0, HALF) + 1
        plan.append((j, n, 'B', max(q0, HALF) - HALF))
        j += n
    while j <= 1024:
        n = min(64, 1025 - j)
        plan.append((j, n, 'R', 440))
        j += n
    return plan


def _body(chunk, tv_hbm, th_hbm, evi_hbm, ehi_hbm, out_hbm,
          iv, ih, tvc, thc, wnd, head, cbuf, sem):
    cc = lax.axis_index("c")
    ss = lax.axis_index("s")
    ri = ss * 2 + cc

    pltpu.sync_copy(evi_hbm, iv)
    pltpu.sync_copy(ehi_hbm, ih)
    pltpu.async_copy(tv_hbm.at[iv], tvc, sem).wait()
    pltpu.async_copy(th_hbm.at[ih], thc, sem).wait()

    cst = [tvc[29, _sl(k)] + thc[29, _sl(k)] for k in range(NV)]

    def head_body(t, carry):
        cv = CLIP - jnp.minimum(t, CLIP)
        vb = [tvc[cv, _sl(k)] for k in range(NV)]
        base = t * 8
        for k in range(NV):
            head[base, _sl(k)] = cst[k]
        for u in range(7):
            ce = jnp.clip(u - ri, -CLIP, CLIP) + CLIP
            for k in range(NV):
                head[base + 1 + u, _sl(k)] = vb[k] + thc[ce, _sl(k)]
        return carry

    lax.fori_loop(0, NHEAD, head_body, 0)

    def cb_body(r, carry):
        for k in range(NV):
            cbuf[r, _sl(k)] = cst[k]
        return carry

    lax.fori_loop(0, CROWS, cb_body, 0)

    def mk_wnd_body(base):
        def wnd_body(q, carry):
            qq = q - 25
            cv = jnp.clip(lax.shift_right_arithmetic(qq, 5) - CLIP,
                          -CLIP, CLIP) + CLIP
            ce = jnp.clip(lax.bitwise_and(qq, 31) - ri, -CLIP, CLIP) + CLIP
            r = q - base
            for k in range(NV):
                wnd[r, _sl(k)] = tvc[cv, _sl(k)] + thc[ce, _sl(k)]
            return carry
        return wnd_body

    descs = []

    def push(d):
        descs.append(d)
        if len(descs) > MAXQ:
            descs.pop(0).wait()

    def drain():
        while descs:
            descs.pop(0).wait()

    bis = range(8 * chunk, 8 * chunk + 8)
    loc = (1 if chunk == 0 else 0) - 256 * chunk
    plans = {bi: _emit_plan(bi) for bi in bis}

    for phase, (base, hi) in enumerate([(0, HALF), (HALF, WROWS)]):
        lax.fori_loop(base, hi, mk_wnd_body(base), 0)
        if phase == 0:
            for bi in bis:
                row = S * bi + ri + loc
                push(pltpu.async_copy(
                    head.at[pl.ds(8 * min(bi, CLIP), 8)],
                    out_hbm.at[row, pl.ds(0, 8)], sem))
        for bi in bis:
            row = S * bi + ri + loc
            for (dst_j, n, kind, off) in plans[bi]:
                if phase == 0 and kind in ('L', 'A'):
                    push(pltpu.async_copy(
                        wnd.at[pl.ds(off, n)],
                        out_hbm.at[row, pl.ds(dst_j, n)], sem))
                elif phase == 1 and kind in ('B', 'R'):
                    push(pltpu.async_copy(
                        wnd.at[pl.ds(off, n)],
                        out_hbm.at[row, pl.ds(dst_j, n)], sem))
        if phase == 0 and chunk == 0:
            @pl.when(ri == 0)
            def _():
                descs0 = []

                def push0(d):
                    descs0.append(d)
                    if len(descs0) > MAXQ:
                        descs0.pop(0).wait()

                full = L // CROWS
                for t in range(full):
                    push0(pltpu.async_copy(
                        cbuf.at[pl.ds(0, CROWS)],
                        out_hbm.at[0, pl.ds(t * CROWS, CROWS)], sem))
                rem = L - full * CROWS
                push0(pltpu.async_copy(
                    cbuf.at[pl.ds(0, rem)],
                    out_hbm.at[0, pl.ds(full * CROWS, rem)], sem))
                for d in descs0:
                    d.wait()
        drain()


@jax.jit
def kernel(length_q, length_k, embeddings_table_v, embeddings_table_h):
    zq = (jnp.asarray(length_q) - L).astype(jnp.int32)
    zk = (jnp.asarray(length_k) - L).astype(jnp.int32)
    t = jnp.arange(32, dtype=jnp.int32)
    evi = jnp.where(t < 29, t + 1, 0) + zq
    ehi = jnp.where(t < 29, t + 1, 0) + zk
    tv128 = jnp.pad(embeddings_table_v, ((0, 0), (0, D)))
    th128 = jnp.pad(embeddings_table_h, ((0, 0), (0, D)))

    mesh = plsc.VectorSubcoreMesh(core_axis_name="c", subcore_axis_name="s")
    parts = []
    for chunk in range(4):
        nrows = 257 if chunk == 0 else 256
        run = pl.kernel(
            functools.partial(_body, chunk),
            out_type=jax.ShapeDtypeStruct((nrows, L, D), jnp.float32),
            mesh=mesh,
            compiler_params=pltpu.CompilerParams(use_tc_tiling_on_sc=True),
            name=f"relpos2d_chunk{chunk}",
            scratch_types=[
                pltpu.VMEM((32,), jnp.int32),
                pltpu.VMEM((32,), jnp.int32),
                pltpu.VMEM((32, 2 * D), jnp.float32),
                pltpu.VMEM((32, 2 * D), jnp.float32),
                pltpu.VMEM((HALF, D), jnp.float32),
                pltpu.VMEM((NHEAD * 8, D), jnp.float32),
                pltpu.VMEM((CROWS, D), jnp.float32),
                pltpu.SemaphoreType.DMA,
            ],
        )
        parts.append(run(tv128, th128, evi, ehi))
    return jnp.concatenate(parts, axis=0)


# speedup vs baseline: 1.2762x; 1.2762x over previous
import functools

import jax
import jax.numpy as jnp
from jax import lax
from jax.experimental import pallas as pl
from jax.experimental.pallas import tpu as pltpu, tpu_sc as plsc

L = 1025
S = 32
NB = 32
D = 64
NLANE = 16
NV = D // NLANE
CLIP = 14
WROWS = 1016
HALF = 512
CROWS = 24
NHEAD = 15
MAXQ = 14


def _sl(k):
    return pl.ds(k * NLANE, NLANE)


def _emit_plan(bi):
    plan = []
    lext = max(0, 32 * (bi - 15))
    j = 8
    while lext > 0:
        plan.append((j, 32, 'L', 0))
        j += 32
        lext -= 32
    j_end = min(1024, 479 + 32 * bi)
    q0 = j + 472 - 32 * bi
    q1 = j_end + 472 - 32 * bi
    if q0 <= min(q1, HALF - 1):
        n = min(q1, HALF - 1) - q0 + 1
        plan.append((j, n, 'A', q0))
        j += n
    if max(q0, HALF) <= q1:
        n = q1 - max(q0, HALF) + 1
        plan.append((j, n, 'B', max(q0, HALF) - HALF))
        j += n
    while j <= 1024:
        n = min(64, 1025 - j)
        plan.append((j, n, 'R', 440))
        j += n
    return plan


def _body(tv_hbm, th_hbm, evi_hbm, ehi_hbm, out0_hbm, out1_hbm,
          iv, ih, tvc, thc, wnd, head, cbuf, sem):
    cc = lax.axis_index("c")
    ss = lax.axis_index("s")
    ri = ss * 2 + cc

    pltpu.sync_copy(evi_hbm, iv)
    pltpu.sync_copy(ehi_hbm, ih)
    pltpu.async_copy(tv_hbm.at[iv], tvc, sem).wait()
    pltpu.async_copy(th_hbm.at[ih], thc, sem).wait()

    cst = [tvc[29, _sl(k)] + thc[29, _sl(k)] for k in range(NV)]

    def head_body(t, carry):
        cv = CLIP - jnp.minimum(t, CLIP)
        vb = [tvc[cv, _sl(k)] for k in range(NV)]
        base = t * 8
        for k in range(NV):
            head[base, _sl(k)] = cst[k]
        for u in range(7):
            ce = jnp.clip(u - ri, -CLIP, CLIP) + CLIP
            for k in range(NV):
                head[base + 1 + u, _sl(k)] = vb[k] + thc[ce, _sl(k)]
        return carry

    lax.fori_loop(0, NHEAD, head_body, 0)

    def cb_body(r, carry):
        for k in range(NV):
            cbuf[r, _sl(k)] = cst[k]
        return carry

    lax.fori_loop(0, CROWS, cb_body, 0)

    def mk_wnd_body(base):
        def wnd_body(q, carry):
            qq = q - 25
            cv = jnp.clip(lax.shift_right_arithmetic(qq, 5) - CLIP,
                          -CLIP, CLIP) + CLIP
            ce = jnp.clip(lax.bitwise_and(qq, 31) - ri, -CLIP, CLIP) + CLIP
            r = q - base
            for k in range(NV):
                wnd[r, _sl(k)] = tvc[cv, _sl(k)] + thc[ce, _sl(k)]
            return carry
        return wnd_body

    descs = []

    def push(d):
        descs.append(d)
        if len(descs) > MAXQ:
            descs.pop(0).wait()

    def drain():
        while descs:
            descs.pop(0).wait()

    def _dst(bi):
        if bi < 16:
            return out0_hbm, 1 + S * bi + ri
        return out1_hbm, S * (bi - 16) + ri

    plans = [_emit_plan(bi) for bi in range(NB)]

    for phase, (base, hi) in enumerate([(0, HALF), (HALF, WROWS)]):
        lax.fori_loop(base, hi, mk_wnd_body(base), 0)
        if phase == 0:
            for bi in range(NB):
                ref, row = _dst(bi)
                push(pltpu.async_copy(
                    head.at[pl.ds(8 * min(bi, CLIP), 8)],
                    ref.at[row, pl.ds(0, 8)], sem))
        for bi in range(NB):
            ref, row = _dst(bi)
            for (dst_j, n, kind, off) in plans[bi]:
                if phase == 0 and kind in ('L', 'A'):
                    push(pltpu.async_copy(
                        wnd.at[pl.ds(off, n)],
                        ref.at[row, pl.ds(dst_j, n)], sem))
                elif phase == 1 and kind in ('B', 'R'):
                    push(pltpu.async_copy(
                        wnd.at[pl.ds(off, n)],
                        ref.at[row, pl.ds(dst_j, n)], sem))
        if phase == 0:
            @pl.when(ri == 0)
            def _():
                descs0 = []

                def push0(d):
                    descs0.append(d)
                    if len(descs0) > MAXQ:
                        descs0.pop(0).wait()

                full = L // CROWS
                for t in range(full):
                    push0(pltpu.async_copy(
                        cbuf.at[pl.ds(0, CROWS)],
                        out0_hbm.at[0, pl.ds(t * CROWS, CROWS)], sem))
                rem = L - full * CROWS
                push0(pltpu.async_copy(
                    cbuf.at[pl.ds(0, rem)],
                    out0_hbm.at[0, pl.ds(full * CROWS, rem)], sem))
                for d in descs0:
                    d.wait()
        drain()


@jax.jit
def kernel(length_q, length_k, embeddings_table_v, embeddings_table_h):
    zq = (jnp.asarray(length_q) - L).astype(jnp.int32)
    zk = (jnp.asarray(length_k) - L).astype(jnp.int32)
    t = jnp.arange(32, dtype=jnp.int32)
    evi = jnp.where(t < 29, t + 1, 0) + zq
    ehi = jnp.where(t < 29, t + 1, 0) + zk
    tv128 = jnp.pad(embeddings_table_v, ((0, 0), (0, D)))
    th128 = jnp.pad(embeddings_table_h, ((0, 0), (0, D)))

    mesh = plsc.VectorSubcoreMesh(core_axis_name="c", subcore_axis_name="s")
    run = pl.kernel(
        _body,
        out_type=[jax.ShapeDtypeStruct((513, L, D), jnp.float32),
                  jax.ShapeDtypeStruct((512, L, D), jnp.float32)],
        mesh=mesh,
        compiler_params=pltpu.CompilerParams(use_tc_tiling_on_sc=True),
        name="relpos2d",
        scratch_types=[
            pltpu.VMEM((32,), jnp.int32),
            pltpu.VMEM((32,), jnp.int32),
            pltpu.VMEM((32, 2 * D), jnp.float32),
            pltpu.VMEM((32, 2 * D), jnp.float32),
            pltpu.VMEM((HALF, D), jnp.float32),
            pltpu.VMEM((NHEAD * 8, D), jnp.float32),
            pltpu.VMEM((CROWS, D), jnp.float32),
            pltpu.SemaphoreType.DMA,
        ],
    )
    o0, o1 = run(tv128, th128, evi, ehi)
    return jnp.concatenate([o0, o1], axis=0)


# speedup vs baseline: 1.6718x; 1.3099x over previous
import jax
import jax.numpy as jnp
from jax import lax
from jax.experimental import pallas as pl
from jax.experimental.pallas import tpu as pltpu, tpu_sc as plsc

L = 1025
S = 32
NB = 32
D = 64
NLANE = 16
NV = D // NLANE
CLIP = 14
WROWS = 1016
HALF = 512
CROWS = 24
NHEAD = 15
MAXQ = 24


def _sl(k):
    return pl.ds(k * NLANE, NLANE)


def _emit_plan(bi):
    plan = []
    lext = max(0, 32 * (bi - 15))
    j = 8
    while lext > 0:
        plan.append((j, 32, 'L', 0))
        j += 32
        lext -= 32
    j_end = min(1024, 479 + 32 * bi)
    q0 = j + 472 - 32 * bi
    q1 = j_end + 472 - 32 * bi
    if q0 <= min(q1, HALF - 1):
        n = min(q1, HALF - 1) - q0 + 1
        plan.append((j, n, 'A', q0))
        j += n
    if max(q0, HALF) <= q1:
        n = q1 - max(q0, HALF) + 1
        plan.append((j, n, 'B', max(q0, HALF) - HALF))
        j += n
    while j <= 1024:
        n = min(64, 1025 - j)
        plan.append((j, n, 'R', 440))
        j += n
    return plan


def _body(tv_hbm, th_hbm, evi_hbm, ehi_hbm, out_hbm,
          iv, ih, tvc, thc, wnd, head, cbuf, sem):
    cc = lax.axis_index("c")
    ss = lax.axis_index("s")
    ri = ss * 2 + cc

    pltpu.sync_copy(evi_hbm, iv)
    pltpu.sync_copy(ehi_hbm, ih)
    pltpu.async_copy(tv_hbm.at[iv], tvc, sem).wait()
    pltpu.async_copy(th_hbm.at[ih], thc, sem).wait()

    cst = [tvc[29, _sl(k)] + thc[29, _sl(k)] for k in range(NV)]

    def head_body(t, carry):
        cv = CLIP - jnp.minimum(t, CLIP)
        vb = [tvc[cv, _sl(k)] for k in range(NV)]
        base = t * 8
        for k in range(NV):
            head[base, _sl(k)] = cst[k]
        for u in range(7):
            ce = jnp.clip(u - ri, -CLIP, CLIP) + CLIP
            for k in range(NV):
                head[base + 1 + u, _sl(k)] = vb[k] + thc[ce, _sl(k)]
        return carry

    lax.fori_loop(0, NHEAD, head_body, 0)

    def cb_body(r, carry):
        for k in range(NV):
            cbuf[r, _sl(k)] = cst[k]
        return carry

    lax.fori_loop(0, CROWS, cb_body, 0)

    def mk_wnd_body(base):
        def wnd_body(q, carry):
            qq = q - 25
            cv = jnp.clip(lax.shift_right_arithmetic(qq, 5) - CLIP,
                          -CLIP, CLIP) + CLIP
            ce = jnp.clip(lax.bitwise_and(qq, 31) - ri, -CLIP, CLIP) + CLIP
            r = q - base
            for k in range(NV):
                wnd[r, _sl(k)] = tvc[cv, _sl(k)] + thc[ce, _sl(k)]
            return carry
        return wnd_body

    descs = []

    def push(d):
        descs.append(d)
        if len(descs) > MAXQ:
            descs.pop(0).wait()

    def drain():
        while descs:
            descs.pop(0).wait()

    plans = [_emit_plan(bi) for bi in range(NB)]

    for phase, (base, hi) in enumerate([(0, HALF), (HALF, WROWS)]):
        lax.fori_loop(base, hi, mk_wnd_body(base), 0)
        if phase == 0:
            for bi in range(NB):
                row = 1 + S * bi + ri
                push(pltpu.async_copy(
                    head.at[pl.ds(8 * min(bi, CLIP), 8)],
                    out_hbm.at[row, pl.ds(0, 8)], sem))
        for bi in range(NB):
            row = 1 + S * bi + ri
            for (dst_j, n, kind, off) in plans[bi]:
                if phase == 0 and kind in ('L', 'A'):
                    push(pltpu.async_copy(
                        wnd.at[pl.ds(off, n)],
                        out_hbm.at[row, pl.ds(dst_j, n)], sem))
                elif phase == 1 and kind in ('B', 'R'):
                    push(pltpu.async_copy(
                        wnd.at[pl.ds(off, n)],
                        out_hbm.at[row, pl.ds(dst_j, n)], sem))
        if phase == 0:
            @pl.when(ri == 0)
            def _():
                descs0 = []

                def push0(d):
                    descs0.append(d)
                    if len(descs0) > MAXQ:
                        descs0.pop(0).wait()

                full = L // CROWS
                for t in range(full):
                    push0(pltpu.async_copy(
                        cbuf.at[pl.ds(0, CROWS)],
                        out_hbm.at[0, pl.ds(t * CROWS, CROWS)], sem))
                rem = L - full * CROWS
                push0(pltpu.async_copy(
                    cbuf.at[pl.ds(0, rem)],
                    out_hbm.at[0, pl.ds(full * CROWS, rem)], sem))
                for d in descs0:
                    d.wait()
        drain()


@jax.jit
def kernel(length_q, length_k, embeddings_table_v, embeddings_table_h):
    zq = (jnp.asarray(length_q) - L).astype(jnp.int32)
    zk = (jnp.asarray(length_k) - L).astype(jnp.int32)
    t = jnp.arange(32, dtype=jnp.int32)
    evi = jnp.where(t < 29, t + 1, 0) + zq
    ehi = jnp.where(t < 29, t + 1, 0) + zk
    tv128 = jnp.pad(embeddings_table_v, ((0, 0), (0, D)))
    th128 = jnp.pad(embeddings_table_h, ((0, 0), (0, D)))

    mesh = plsc.VectorSubcoreMesh(core_axis_name="c", subcore_axis_name="s")
    run = pl.kernel(
        _body,
        out_type=jax.ShapeDtypeStruct((L, L, D), jnp.float32),
        mesh=mesh,
        compiler_params=pltpu.CompilerParams(use_tc_tiling_on_sc=True),
        scratch_types=[
            pltpu.VMEM((32,), jnp.int32),
            pltpu.VMEM((32,), jnp.int32),
            pltpu.VMEM((32, 2 * D), jnp.float32),
            pltpu.VMEM((32, 2 * D), jnp.float32),
            pltpu.VMEM((HALF, D), jnp.float32),
            pltpu.VMEM((NHEAD * 8, D), jnp.float32),
            pltpu.VMEM((CROWS, D), jnp.float32),
            pltpu.SemaphoreType.DMA,
        ],
    )
    return run(tv128, th128, evi, ehi)


# speedup vs baseline: 1.6723x; 1.0003x over previous
import jax
import jax.numpy as jnp
from jax import lax
from jax.experimental import pallas as pl
from jax.experimental.pallas import tpu as pltpu, tpu_sc as plsc

L = 1025
S = 32
NB = 32
D = 64
NLANE = 16
NV = D // NLANE
CLIP = 14
WROWS = 1016
HALF = 512
CROWS = 24
NHEAD = 15
MAXQ = 24


def _sl(k):
    return pl.ds(k * NLANE, NLANE)


def _emit_plan(bi):
    plan = []
    lext = max(0, 32 * (bi - 15))
    j = 8
    while lext > 0:
        plan.append((j, 32, 'L', 0))
        j += 32
        lext -= 32
    j_end = min(1024, 479 + 32 * bi)
    q0 = j + 472 - 32 * bi
    q1 = j_end + 472 - 32 * bi
    if q0 <= min(q1, HALF - 1):
        n = min(q1, HALF - 1) - q0 + 1
        plan.append((j, n, 'A', q0))
        j += n
    if max(q0, HALF) <= q1:
        n = q1 - max(q0, HALF) + 1
        plan.append((j, n, 'B', max(q0, HALF) - HALF))
        j += n
    while j <= 1024:
        n = min(64, 1025 - j)
        plan.append((j, n, 'R', 440))
        j += n
    return plan


def _body(tv_hbm, th_hbm, evi_hbm, ehi_hbm, out_hbm,
          iv, ih, tvc, thc, wnd, head, cbuf, sem):
    cc = lax.axis_index("c")
    ss = lax.axis_index("s")
    ri = ss * 2 + cc

    pltpu.sync_copy(evi_hbm, iv)
    pltpu.sync_copy(ehi_hbm, ih)
    pltpu.async_copy(tv_hbm.at[iv], tvc, sem).wait()
    pltpu.async_copy(th_hbm.at[ih], thc, sem).wait()

    cst = [tvc[29, _sl(k)] + thc[29, _sl(k)] for k in range(NV)]

    def head_body(t, carry):
        cv = CLIP - jnp.minimum(t, CLIP)
        vb = [tvc[cv, _sl(k)] for k in range(NV)]
        base = t * 8
        for k in range(NV):
            head[base, _sl(k)] = cst[k]
        for u in range(7):
            ce = jnp.clip(u - ri, -CLIP, CLIP) + CLIP
            for k in range(NV):
                head[base + 1 + u, _sl(k)] = vb[k] + thc[ce, _sl(k)]
        return carry

    lax.fori_loop(0, NHEAD, head_body, 0)

    def cb_body(r, carry):
        for k in range(NV):
            cbuf[r, _sl(k)] = cst[k]
        return carry

    lax.fori_loop(0, CROWS, cb_body, 0)

    def mk_wnd_body(base):
        def wnd_body(q, carry):
            qq = q - 25
            cv = jnp.clip(lax.shift_right_arithmetic(qq, 5) - CLIP,
                          -CLIP, CLIP) + CLIP
            ce = jnp.clip(lax.bitwise_and(qq, 31) - ri, -CLIP, CLIP) + CLIP
            r = q - base
            for k in range(NV):
                wnd[r, _sl(k)] = tvc[cv, _sl(k)] + thc[ce, _sl(k)]
            return carry
        return wnd_body

    descs = []

    def push(d):
        descs.append(d)
        if len(descs) > MAXQ:
            descs.pop(0).wait()

    def drain():
        while descs:
            descs.pop(0).wait()

    plans = [_emit_plan(bi) for bi in range(NB)]

    for phase, (base, hi) in enumerate([(0, HALF), (HALF, WROWS)]):
        lax.fori_loop(base, hi, mk_wnd_body(base), 0)
        if phase == 0:
            for bi in range(NB):
                row = 1 + S * bi + ri
                push(pltpu.async_copy(
                    head.at[pl.ds(8 * min(bi, CLIP), 8)],
                    out_hbm.at[row, pl.ds(0, 8)], sem))
        for bi in range(NB):
            row = 1 + S * bi + ri
            for (dst_j, n, kind, off) in plans[bi]:
                if phase == 0 and kind in ('L', 'A'):
                    push(pltpu.async_copy(
                        wnd.at[pl.ds(off, n)],
                        out_hbm.at[row, pl.ds(dst_j, n)], sem))
                elif phase == 1 and kind in ('B', 'R'):
                    push(pltpu.async_copy(
                        wnd.at[pl.ds(off, n)],
                        out_hbm.at[row, pl.ds(dst_j, n)], sem))
        if phase == 0:
            @pl.when(ri == 0)
            def _():
                descs0 = []

                def push0(d):
                    descs0.append(d)
                    if len(descs0) > MAXQ:
                        descs0.pop(0).wait()

                full = L // CROWS
                for t in range(full):
                    push0(pltpu.async_copy(
                        cbuf.at[pl.ds(0, CROWS)],
                        out_hbm.at[0, pl.ds(t * CROWS, CROWS)], sem))
                rem = L - full * CROWS
                push0(pltpu.async_copy(
                    cbuf.at[pl.ds(0, rem)],
                    out_hbm.at[0, pl.ds(full * CROWS, rem)], sem))
                for d in descs0:
                    d.wait()
        drain()


@jax.jit
def kernel(length_q, length_k, embeddings_table_v, embeddings_table_h):
    zq = (jnp.asarray(length_q) - L).astype(jnp.int32)
    zk = (jnp.asarray(length_k) - L).astype(jnp.int32)
    t = jnp.arange(32, dtype=jnp.int32)
    evi = jnp.where(t < 29, t + 1, 0) + zq
    ehi = jnp.where(t < 29, t + 1, 0) + zk
    tv128 = jnp.pad(embeddings_table_v, ((0, 0), (0, D)))
    th128 = jnp.pad(embeddings_table_h, ((0, 0), (0, D)))

    mesh = plsc.VectorSubcoreMesh(core_axis_name="c", subcore_axis_name="s")
    run = pl.kernel(
        _body,
        out_type=jax.ShapeDtypeStruct((L, L, D), jnp.float32),
        mesh=mesh,
        compiler_params=pltpu.CompilerParams(use_tc_tiling_on_sc=True,
                                             has_side_effects=False),
        scratch_types=[
            pltpu.VMEM((32,), jnp.int32),
            pltpu.VMEM((32,), jnp.int32),
            pltpu.VMEM((32, 2 * D), jnp.float32),
            pltpu.VMEM((32, 2 * D), jnp.float32),
            pltpu.VMEM((HALF, D), jnp.float32),
            pltpu.VMEM((NHEAD * 8, D), jnp.float32),
            pltpu.VMEM((CROWS, D), jnp.float32),
            pltpu.SemaphoreType.DMA,
        ],
    )
    return run(tv128, th128, evi, ehi)
